# baseline (speedup 1.0000x reference)
import jax
import jax.numpy as jnp
from jax import lax
from jax.experimental import pallas as pl
from jax.experimental.pallas import tpu as pltpu
from jax.experimental.pallas import tpu_sc as plsc

VOCAB = 100000
N_EMBD = 128
N_CTX = 2048
BATCH = 64
SEQ = 2048

NC = 2
NS = 16
NW = NC * NS
LANES = 16

POS_PER_W = SEQ // NW
NCHUNK = BATCH
NBUF = 3
SPLIT = 32


def _embed_body(ids_hbm, tok_hbm, pos_hbm, out_hbm,
                idx_all, pos_v, iota_v, rows0, rows1, rows2, pos_sp,
                gs0, gs1, gs2, ps0, ps1, ps2, ws0, ws1, ws2):
    cid = lax.axis_index("c")
    sid = lax.axis_index("s")
    wid = sid * NC + cid
    rows = (rows0, rows1, rows2)
    gsem = (gs0, gs1, gs2)
    psem = (ps0, ps1, ps2)
    wsem = (ws0, ws1, ws2)

    pltpu.sync_copy(ids_hbm.at[:, wid], idx_all)
    pltpu.sync_copy(pos_hbm.at[pl.ds(wid * POS_PER_W, POS_PER_W)], pos_v)
    my_pos_sp = pos_sp.at[pl.ds(sid * POS_PER_W, SPLIT)]
    pltpu.sync_copy(pos_v.at[pl.ds(0, SPLIT)], my_pos_sp)
    for t in range(SPLIT // LANES):
        iota_v[pl.ds(t * LANES, LANES)] = lax.iota(jnp.int32, LANES) + t * LANES

    def fire_gather(b, k):
        pltpu.make_async_copy(tok_hbm.at[idx_all.at[b]], rows[k], gsem[k]).start()

    def wait_gather(b, k):
        pltpu.make_async_copy(tok_hbm.at[idx_all.at[b]], rows[k], gsem[k]).wait()

    def fire_posadd(k):
        pltpu.make_async_copy(my_pos_sp.at[iota_v], rows[k].at[pl.ds(0, SPLIT)],
                              psem[k]).start(add=True)

    def wait_posadd(k):
        pltpu.make_async_copy(my_pos_sp.at[iota_v], rows[k].at[pl.ds(0, SPLIT)],
                              psem[k]).wait()

    def vec_add(k):
        @plsc.parallel_loop(SPLIT, POS_PER_W, unroll=4)
        def _(i):
            for j in range(N_EMBD // LANES):
                sl = pl.ds(j * LANES, LANES)
                plsc.addupdate(rows[k].at[i, sl], pos_v[i, sl])

    def fire_write(b, k):
        pltpu.make_async_copy(rows[k], out_hbm.at[b, wid], wsem[k]).start()

    def wait_write(b, k):
        pltpu.make_async_copy(rows[k], out_hbm.at[b, wid], wsem[k]).wait()

    fire_gather(0, 0)

    fire_gather(1, 1)
    wait_gather(0, 0)
    fire_posadd(0)
    vec_add(0)

    fire_gather(2, 2)
    wait_gather(1, 1)
    fire_posadd(1)
    vec_add(1)
    wait_posadd(0)
    fire_write(0, 0)

    wait_write(0, 0)
    fire_gather(3, 0)
    wait_gather(2, 2)
    fire_posadd(2)
    vec_add(2)
    wait_posadd(1)
    fire_write(1, 1)

    def group(g, _):
        for u in range(NBUF):
            b = g * NBUF + u
            k = u
            kn = (u + 1) % NBUF
            kp = (u + 2) % NBUF
            wait_write(b - 2, kn)
            fire_gather(b + 1, kn)
            wait_gather(b, k)
            fire_posadd(k)
            vec_add(k)
            wait_posadd(kp)
            fire_write(b - 1, kp)
        return 0

    lax.fori_loop(1, (NCHUNK - 1) // NBUF, group, 0)

    b = NCHUNK - 1
    wait_write(b - 2, 1)
    wait_gather(b, 0)
    fire_posadd(0)
    vec_add(0)
    wait_posadd(2)
    fire_write(b - 1, 2)
    wait_posadd(0)
    fire_write(b, 0)
    wait_write(b - 1, 2)
    wait_write(b, 0)


@jax.jit
def _embed(ids_r, tok_table, pos_table):
    mesh = plsc.VectorSubcoreMesh(core_axis_name="c", subcore_axis_name="s")
    return pl.kernel(
        _embed_body,
        out_type=jax.ShapeDtypeStruct((BATCH, NW, POS_PER_W, N_EMBD), jnp.float32),
        mesh=mesh,
        scratch_types=[
            pltpu.VMEM((NCHUNK, POS_PER_W), jnp.int32),
            pltpu.VMEM((POS_PER_W, N_EMBD), jnp.float32),
            pltpu.VMEM((SPLIT,), jnp.int32),
            pltpu.VMEM((POS_PER_W, N_EMBD), jnp.float32),
            pltpu.VMEM((POS_PER_W, N_EMBD), jnp.float32),
            pltpu.VMEM((POS_PER_W, N_EMBD), jnp.float32),
            pltpu.VMEM_SHARED((NS * POS_PER_W, N_EMBD), jnp.float32),
            pltpu.SemaphoreType.DMA,
            pltpu.SemaphoreType.DMA,
            pltpu.SemaphoreType.DMA,
            pltpu.SemaphoreType.DMA,
            pltpu.SemaphoreType.DMA,
            pltpu.SemaphoreType.DMA,
            pltpu.SemaphoreType.DMA,
            pltpu.SemaphoreType.DMA,
            pltpu.SemaphoreType.DMA,
        ],
    )(ids_r, tok_table, pos_table)


def kernel(ids, tok_table, pos_table):
    B, L = ids.shape
    ids_r = ids.astype(jnp.int32).reshape(B, NW, POS_PER_W)
    out = _embed(ids_r, tok_table, pos_table)
    attn_mask = jnp.ones((B, 1, 1, L), dtype=bool)
    return out.reshape(B, L, N_EMBD), attn_mask


# speedup vs baseline: 1.2236x; 1.2236x over previous
import jax
import jax.numpy as jnp
from jax import lax
from jax.experimental import pallas as pl
from jax.experimental.pallas import tpu as pltpu
from jax.experimental.pallas import tpu_sc as plsc

VOCAB = 100000
N_EMBD = 128
N_CTX = 2048
BATCH = 64
SEQ = 2048

NC = 2
NS = 16
NW = NC * NS
LANES = 16

POS_PER_W = SEQ // NW
BPC = 2
ROWS_PER_C = BPC * POS_PER_W
NCHUNK = BATCH // BPC
NBUF = 3


def _embed_body(ids_hbm, tok_hbm, pos_hbm, out_hbm,
                idx_all, pos_v, iota_v, rows0, rows1, rows2, pos_sp,
                gs0, gs1, gs2, ps0, ps1, ps2, ws0, ws1, ws2):
    cid = lax.axis_index("c")
    sid = lax.axis_index("s")
    wid = sid * NC + cid
    rows = (rows0, rows1, rows2)
    gsem = (gs0, gs1, gs2)
    psem = (ps0, ps1, ps2)
    wsem = (ws0, ws1, ws2)

    pltpu.sync_copy(ids_hbm.at[wid], idx_all)
    pltpu.sync_copy(pos_hbm.at[pl.ds(wid * POS_PER_W, POS_PER_W)], pos_v)
    my_pos_sp = pos_sp.at[pl.ds(sid * POS_PER_W, POS_PER_W)]
    pltpu.sync_copy(pos_v, my_pos_sp)
    for t in range(ROWS_PER_C // LANES):
        iota_v[pl.ds(t * LANES, LANES)] = (
            lax.iota(jnp.int32, LANES) + (t * LANES) % POS_PER_W
        )

    def fire_gather(c, k):
        pltpu.make_async_copy(tok_hbm.at[idx_all.at[c]], rows[k], gsem[k]).start()

    def wait_gather(c, k):
        pltpu.make_async_copy(tok_hbm.at[idx_all.at[c]], rows[k], gsem[k]).wait()

    def fire_posadd(k):
        pltpu.make_async_copy(my_pos_sp.at[iota_v], rows[k],
                              psem[k]).start(add=True)

    def wait_posadd(k):
        pltpu.make_async_copy(my_pos_sp.at[iota_v], rows[k], psem[k]).wait()

    def fire_write(c, k):
        for h in range(BPC):
            pltpu.make_async_copy(
                rows[k].at[pl.ds(h * POS_PER_W, POS_PER_W)],
                out_hbm.at[c * BPC + h, wid], wsem[k]).start()

    def wait_write(c, k):
        for h in range(BPC):
            pltpu.make_async_copy(
                rows[k].at[pl.ds(h * POS_PER_W, POS_PER_W)],
                out_hbm.at[c * BPC + h, wid], wsem[k]).wait()

    fire_gather(0, 0)

    fire_gather(1, 1)
    wait_gather(0, 0)
    fire_posadd(0)

    fire_gather(2, 2)
    wait_gather(1, 1)
    fire_posadd(1)
    wait_posadd(0)
    fire_write(0, 0)

    wait_write(0, 0)
    fire_gather(3, 0)
    wait_gather(2, 2)
    fire_posadd(2)
    wait_posadd(1)
    fire_write(1, 1)

    def group(g, _):
        for u in range(NBUF):
            c = g * NBUF + u
            k = u
            kn = (u + 1) % NBUF
            kp = (u + 2) % NBUF
            wait_write(c - 2, kn)
            fire_gather(c + 1, kn)
            wait_gather(c, k)
            fire_posadd(k)
            wait_posadd(kp)
            fire_write(c - 1, kp)
        return 0

    lax.fori_loop(1, (NCHUNK - 2) // NBUF, group, 0)

    c = NCHUNK - 2
    wait_write(c - 2, 1)
    fire_gather(c + 1, 1)
    wait_gather(c, 0)
    fire_posadd(0)
    wait_posadd(2)
    fire_write(c - 1, 2)

    c = NCHUNK - 1
    wait_write(c - 2, 2)
    wait_gather(c, 1)
    fire_posadd(1)
    wait_posadd(0)
    fire_write(c - 1, 0)
    wait_posadd(1)
    fire_write(c, 1)
    wait_write(c - 1, 0)
    wait_write(c, 1)


@jax.jit
def _embed(ids_r, tok_table, pos_table):
    mesh = plsc.VectorSubcoreMesh(core_axis_name="c", subcore_axis_name="s")
    return pl.kernel(
        _embed_body,
        out_type=jax.ShapeDtypeStruct((BATCH, NW, POS_PER_W, N_EMBD), jnp.float32),
        mesh=mesh,
        scratch_types=[
            pltpu.VMEM((NCHUNK, ROWS_PER_C), jnp.int32),
            pltpu.VMEM((POS_PER_W, N_EMBD), jnp.float32),
            pltpu.VMEM((ROWS_PER_C,), jnp.int32),
            pltpu.VMEM((ROWS_PER_C, N_EMBD), jnp.float32),
            pltpu.VMEM((ROWS_PER_C, N_EMBD), jnp.float32),
            pltpu.VMEM((ROWS_PER_C, N_EMBD), jnp.float32),
            pltpu.VMEM_SHARED((NS * POS_PER_W, N_EMBD), jnp.float32),
            pltpu.SemaphoreType.DMA,
            pltpu.SemaphoreType.DMA,
            pltpu.SemaphoreType.DMA,
            pltpu.SemaphoreType.DMA,
            pltpu.SemaphoreType.DMA,
            pltpu.SemaphoreType.DMA,
            pltpu.SemaphoreType.DMA,
            pltpu.SemaphoreType.DMA,
            pltpu.SemaphoreType.DMA,
        ],
    )(ids_r, tok_table, pos_table)


def kernel(ids, tok_table, pos_table):
    B, L = ids.shape
    ids_r = (ids.astype(jnp.int32).reshape(B, NW, POS_PER_W)
             .transpose(1, 0, 2).reshape(NW, NCHUNK, ROWS_PER_C))
    out = _embed(ids_r, tok_table, pos_table)
    attn_mask = jnp.ones((B, 1, 1, L), dtype=bool)
    return out.reshape(B, L, N_EMBD), attn_mask
